# Initial kernel scaffold; baseline (speedup 1.0000x reference)
import jax
import jax.numpy as jnp
from jax.experimental import pallas as pl


def kernel(x, edge_index, W1, b1, gw1, W2, b2, gw2, W3, b3):
    raise NotImplementedError("write your pallas kernel here")


# speedup vs baseline: 11.2663x; 11.2663x over previous
import functools

import jax
import jax.numpy as jnp
from jax import lax
from jax.experimental import pallas as pl
from jax.experimental.pallas import tpu as pltpu
from jax.experimental.pallas import tpu_sc as plsc

_N = 10000
_E = 320000
_D = 128
_PD = 8
_NS = 16
_EC = 128
_NCH = 160
_ET = _NCH * _EC
_EP = _NS * _ET
_NP = 10112
_RPT = _NP // _NS


def _q_body(x_ref, w1_ref, w2_ref, gw1c_ref, w3p_ref, gw2c_ref, b1r_ref,
            o_ref):
    w2p = w2_ref[...] * gw1c_ref[...]
    w3p = w3p_ref[...] * gw2c_ref[...]
    w23 = jnp.dot(w2p, w3p, preferred_element_type=jnp.float32)
    wq = jnp.dot(w1_ref[...], w23, preferred_element_type=jnp.float32)
    cq = jnp.dot(b1r_ref[...], w23, preferred_element_type=jnp.float32)
    ones_col = (lax.broadcasted_iota(jnp.int32, (1, _PD), 1) == 4
                ).astype(jnp.float32)
    o_ref[...] = (jnp.dot(x_ref[...], wq, preferred_element_type=jnp.float32)
                  + cq + ones_col)


def _make_q(xp, W1, W2, gw1c, W3p, gw2c, b1r):
    blk = _NP // 4
    return pl.pallas_call(
        _q_body,
        grid=(4,),
        in_specs=[
            pl.BlockSpec((blk, _D), lambda i: (i, 0)),
            pl.BlockSpec((_D, 96), lambda i: (0, 0)),
            pl.BlockSpec((96, 64), lambda i: (0, 0)),
            pl.BlockSpec((96, 1), lambda i: (0, 0)),
            pl.BlockSpec((64, _PD), lambda i: (0, 0)),
            pl.BlockSpec((64, 1), lambda i: (0, 0)),
            pl.BlockSpec((1, 96), lambda i: (0, 0)),
        ],
        out_specs=pl.BlockSpec((blk, _PD), lambda i: (i, 0)),
        out_shape=jax.ShapeDtypeStruct((_NP, _PD), jnp.float32),
    )(xp, W1, W2, gw1c, W3p, gw2c, b1r)


def _sc_body(q_hbm, row_hbm, col_hbm, zeros_hbm, s1_hbm, s2_hbm,
             rowv, colv, gbuf, acc1, acc2, sem):
    sid = lax.axis_index("s")
    r0 = sid * _RPT
    pltpu.sync_copy(zeros_hbm.at[pl.ds(r0, _RPT)], acc1.at[pl.ds(r0, _RPT)])
    pltpu.sync_copy(zeros_hbm.at[pl.ds(r0, _RPT)], acc2.at[pl.ds(r0, _RPT)])
    pltpu.sync_copy(row_hbm.at[sid], rowv)
    pltpu.sync_copy(col_hbm.at[sid], colv)
    plsc.subcore_barrier()

    def hop1(j, carry):
        pltpu.async_copy(q_hbm.at[colv.at[j]], gbuf, sem).wait()
        pltpu.sync_copy(gbuf, acc1.at[rowv.at[j]], add=True)
        return carry

    lax.fori_loop(0, _NCH, hop1, 0)
    plsc.subcore_barrier()
    pltpu.sync_copy(acc1.at[pl.ds(r0, _RPT)], s1_hbm.at[pl.ds(r0, _RPT)])

    def hop2(j, carry):
        pltpu.async_copy(acc1.at[colv.at[j]], gbuf, sem).wait()
        pltpu.sync_copy(gbuf, acc2.at[rowv.at[j]], add=True)
        return carry

    lax.fori_loop(0, _NCH, hop2, 0)
    plsc.subcore_barrier()
    pltpu.sync_copy(acc2.at[pl.ds(r0, _RPT)], s2_hbm.at[pl.ds(r0, _RPT)])


def _make_sc(qt, rowp, colp, zeros):
    mesh = plsc.VectorSubcoreMesh(core_axis_name="c", subcore_axis_name="s",
                                  num_cores=1)
    f = pl.kernel(
        _sc_body,
        out_type=[jax.ShapeDtypeStruct((_NP, _PD), jnp.float32),
                  jax.ShapeDtypeStruct((_NP, _PD), jnp.float32)],
        mesh=mesh,
        compiler_params=pltpu.CompilerParams(use_tc_tiling_on_sc=False),
        scratch_types=[
            pltpu.VMEM((_NCH, _EC), jnp.int32),
            pltpu.VMEM((_NCH, _EC), jnp.int32),
            pltpu.VMEM((_EC, _PD), jnp.float32),
            pltpu.VMEM_SHARED((_NP, _PD), jnp.float32),
            pltpu.VMEM_SHARED((_NP, _PD), jnp.float32),
            pltpu.SemaphoreType.DMA,
        ],
    )
    return f(qt, rowp, colp, zeros)


def _ep_body(s1_ref, s2_ref, w3p_ref, gw2c_ref, b2r_ref, b3r_ref, o_ref):
    w3p = w3p_ref[...] * gw2c_ref[...]
    c2 = jnp.dot(b2r_ref[...], w3p, preferred_element_type=jnp.float32)
    deg = s1_ref[:, 4:5]
    out8 = s2_ref[...] + deg * c2 + b3r_ref[...]
    o_ref[...] = out8[:_N, :4]


def _make_ep(s1, s2, W3p, gw2c, b2r, b3r):
    return pl.pallas_call(
        _ep_body,
        grid=(1,),
        in_specs=[
            pl.BlockSpec((_NP, _PD), lambda i: (0, 0)),
            pl.BlockSpec((_NP, _PD), lambda i: (0, 0)),
            pl.BlockSpec((64, _PD), lambda i: (0, 0)),
            pl.BlockSpec((64, 1), lambda i: (0, 0)),
            pl.BlockSpec((1, 64), lambda i: (0, 0)),
            pl.BlockSpec((1, _PD), lambda i: (0, 0)),
        ],
        out_specs=pl.BlockSpec((_N, 4), lambda i: (0, 0)),
        out_shape=jax.ShapeDtypeStruct((_N, 4), jnp.float32),
    )(s1, s2, W3p, gw2c, b2r, b3r)


def kernel(x, edge_index, W1, b1, gw1, W2, b2, gw2, W3, b3):
    xp = jnp.pad(x, ((0, _NP - _N), (0, 0)))
    gw1c = gw1.reshape(96, 1)
    gw2c = gw2.reshape(64, 1)
    W3p = jnp.pad(W3, ((0, 0), (0, _PD - 4)))
    b1r = b1.reshape(1, 96)
    b2r = b2.reshape(1, 64)
    b3r = jnp.pad(b3, (0, _PD - 4)).reshape(1, _PD)
    pad_idx = jnp.full((_EP - _E,), _N, dtype=jnp.int32)
    rowp = jnp.concatenate([edge_index[0], pad_idx]).reshape(_NS, _NCH, _EC)
    colp = jnp.concatenate([edge_index[1], pad_idx]).reshape(_NS, _NCH, _EC)
    zeros = jnp.zeros((_NP, _PD), jnp.float32)

    qt = _make_q(xp, W1, W2, gw1c, W3p, gw2c, b1r)
    s1, s2 = _make_sc(qt, rowp, colp, zeros)
    return _make_ep(s1, s2, W3p, gw2c, b2r, b3r)
